# baseline (speedup 1.0000x reference)
import functools

import jax
import jax.numpy as jnp
from jax import lax
from jax.experimental import pallas as pl
from jax.experimental.pallas import tpu as pltpu
from jax.experimental.pallas import tpu_sc as plsc

N_CODES = 8192
DIM = 32
N_INPUTS = 8192
BLK_IN = 4096
BLK_CODE = 1024
N_BLOCKS = N_INPUTS // BLK_IN
N_CHUNKS = N_CODES // BLK_CODE


def _argmin_body(xt_ref, emb_ref, en_ref, xn_ref, idx_ref, loss_ref):
    xt = xt_ref[...]
    xn = xn_ref[0]
    rows = lax.broadcasted_iota(jnp.int32, (BLK_CODE, 1), 0).astype(jnp.float32)

    def tile_minarg(t):
        e2 = emb_ref[pl.ds(t * BLK_CODE, BLK_CODE), :]
        en = en_ref[pl.ds(t * BLK_CODE, BLK_CODE), :]
        mm = lax.dot_general(e2, xt, (((1,), (0,)), ((), ())),
                             preferred_element_type=jnp.float32)
        d2 = (xn + en) + mm
        tmin2 = jnp.min(d2, axis=0, keepdims=True)
        tidx_f = jnp.min(jnp.where(d2 == tmin2, rows, jnp.float32(2**30)),
                         axis=0, keepdims=True)
        tidx = tidx_f.astype(jnp.int32) + t * BLK_CODE
        tmin = jnp.sqrt(jnp.maximum(tmin2, 0.0))
        return tmin, tidx

    def chunk_step(c, carry):
        bq, bi, fv = carry
        cmin, cidx = tile_minarg(2 * c)
        tmin, tidx = tile_minarg(2 * c + 1)
        b = tmin < cmin
        cmin = jnp.where(b, tmin, cmin)
        cidx = jnp.where(b, tidx, cidx)
        upd = cmin < bq
        bq = jnp.where(upd, cmin.astype(jnp.bfloat16).astype(jnp.float32), bq)
        bi = jnp.where(upd, cidx, bi)
        return (bq, bi, jnp.minimum(fv, cmin))

    carry = (jnp.full((1, BLK_IN), jnp.inf, jnp.float32),
             jnp.zeros((1, BLK_IN), jnp.int32),
             jnp.full((1, BLK_IN), jnp.inf, jnp.float32))
    for c in range(N_CHUNKS // 2):
        carry = chunk_step(c, carry)
    _, best_i, best_d = carry
    idx_ref[0] = best_i

    pid = pl.program_id(0)

    @pl.when(pid == 0)
    def _init():
        loss_ref[...] = jnp.zeros((1, 1), jnp.float32)

    loss_ref[...] = loss_ref[...] + jnp.sum(best_d * best_d).reshape(1, 1)

    @pl.when(pid == N_BLOCKS - 1)
    def _finish():
        loss_ref[...] = loss_ref[...] * (1.0 / (N_INPUTS * DIM))


_argmin_call = pl.pallas_call(
    _argmin_body,
    grid=(N_BLOCKS,),
    in_specs=[
        pl.BlockSpec((DIM, BLK_IN), lambda i: (0, i)),
        pl.BlockSpec((N_CODES, DIM), lambda i: (0, 0)),
        pl.BlockSpec((N_CODES, 1), lambda i: (0, 0)),
        pl.BlockSpec((1, 1, BLK_IN), lambda i: (i, 0, 0)),
    ],
    out_specs=[
        pl.BlockSpec((1, 1, BLK_IN), lambda i: (i, 0, 0)),
        pl.BlockSpec((1, 1), lambda i: (0, 0)),
    ],
    out_shape=[
        jax.ShapeDtypeStruct((N_BLOCKS, 1, BLK_IN), jnp.int32),
        jax.ShapeDtypeStruct((1, 1), jnp.float32),
    ],
)


_SC_WORKERS = 32
_ROWS_PER_W = N_INPUTS // _SC_WORKERS


@functools.lru_cache(maxsize=1)
def _make_sc_gather():
    @functools.partial(
        pl.kernel,
        mesh=plsc.VectorSubcoreMesh(core_axis_name="c", subcore_axis_name="s"),
        out_type=jax.ShapeDtypeStruct((N_INPUTS, DIM), jnp.float32),
        scratch_types=[
            pltpu.VMEM((_ROWS_PER_W,), jnp.int32),
            pltpu.VMEM((_ROWS_PER_W, DIM), jnp.float32),
            pltpu.SemaphoreType.DMA,
        ],
        compiler_params=pltpu.CompilerParams(use_tc_tiling_on_sc=False),
    )
    def _sc_gather(table_hbm, idx_hbm, out_hbm, idx_v, rows_v, sem):
        wid = lax.axis_index("s") * 2 + lax.axis_index("c")
        base = wid * _ROWS_PER_W
        pltpu.sync_copy(idx_hbm.at[pl.ds(base, _ROWS_PER_W)], idx_v)
        pltpu.async_copy(table_hbm.at[idx_v], rows_v, sem).wait()
        pltpu.sync_copy(rows_v, out_hbm.at[pl.ds(base, _ROWS_PER_W)])

    return _sc_gather


def kernel(inputs, embeddings):
    input_shape = inputs.shape
    flat = inputs.reshape(-1, DIM)
    xn = jnp.sum(flat ** 2, axis=1)
    en = jnp.sum(embeddings ** 2, axis=1, keepdims=True)
    idx3, loss = _argmin_call(flat.T.astype(jnp.bfloat16), -2.0 * embeddings,
                              en, xn.reshape(N_BLOCKS, 1, BLK_IN))
    idx = idx3.reshape(N_INPUTS)
    quant = _make_sc_gather()(embeddings, idx).reshape(input_shape)
    return (quant, loss[0, 0], idx)


# speedup vs baseline: 1.3045x; 1.3045x over previous
import functools

import jax
import jax.numpy as jnp
from jax import lax
from jax.experimental import pallas as pl
from jax.experimental.pallas import tpu as pltpu
from jax.experimental.pallas import tpu_sc as plsc

N_CODES = 8192
DIM = 32
N_INPUTS = 8192
BLK_IN = 4096
BLK_CODE = 1024
N_BLOCKS = N_INPUTS // BLK_IN
N_CHUNKS = N_CODES // BLK_CODE


def _argmin_body(xt_ref, emb_ref, en_ref, xn_ref, idx_ref, loss_ref):
    xt = xt_ref[...]
    xn = xn_ref[0]
    rows = lax.broadcasted_iota(jnp.int32, (BLK_CODE, 1), 0).astype(jnp.float32)

    def tile_minarg(t):
        e2 = emb_ref[pl.ds(t * BLK_CODE, BLK_CODE), :]
        en = en_ref[pl.ds(t * BLK_CODE, BLK_CODE), :]
        mm = lax.dot_general(e2, xt, (((1,), (0,)), ((), ())),
                             preferred_element_type=jnp.float32)
        d2 = (xn + en) + mm
        tmin2 = jnp.min(d2, axis=0, keepdims=True)
        tidx_f = jnp.min(jnp.where(d2 == tmin2, rows, jnp.float32(2**30)),
                         axis=0, keepdims=True)
        tidx = tidx_f.astype(jnp.int32) + t * BLK_CODE
        tmin = jnp.sqrt(jnp.maximum(tmin2, 0.0))
        return tmin, tidx

    def chunk_step(c, carry):
        bq, bi, fv = carry
        cmin, cidx = tile_minarg(2 * c)
        tmin, tidx = tile_minarg(2 * c + 1)
        b = tmin < cmin
        cmin = jnp.where(b, tmin, cmin)
        cidx = jnp.where(b, tidx, cidx)
        upd = cmin < bq
        bq = jnp.where(upd, cmin.astype(jnp.bfloat16).astype(jnp.float32), bq)
        bi = jnp.where(upd, cidx, bi)
        return (bq, bi, jnp.minimum(fv, cmin))

    init = (jnp.full((1, BLK_IN), jnp.inf, jnp.float32),
            jnp.zeros((1, BLK_IN), jnp.int32),
            jnp.full((1, BLK_IN), jnp.inf, jnp.float32))
    _, best_i, best_d = lax.fori_loop(0, N_CHUNKS // 2, chunk_step, init)
    idx_ref[0] = best_i

    pid = pl.program_id(0)

    @pl.when(pid == 0)
    def _init():
        loss_ref[...] = jnp.zeros((1, 1), jnp.float32)

    loss_ref[...] = loss_ref[...] + jnp.sum(best_d * best_d).reshape(1, 1)

    @pl.when(pid == N_BLOCKS - 1)
    def _finish():
        loss_ref[...] = loss_ref[...] * (1.0 / (N_INPUTS * DIM))


_argmin_call = pl.pallas_call(
    _argmin_body,
    grid=(N_BLOCKS,),
    in_specs=[
        pl.BlockSpec((DIM, BLK_IN), lambda i: (0, i)),
        pl.BlockSpec((N_CODES, DIM), lambda i: (0, 0)),
        pl.BlockSpec((N_CODES, 1), lambda i: (0, 0)),
        pl.BlockSpec((1, 1, BLK_IN), lambda i: (i, 0, 0)),
    ],
    out_specs=[
        pl.BlockSpec((1, 1, BLK_IN), lambda i: (i, 0, 0)),
        pl.BlockSpec((1, 1), lambda i: (0, 0)),
    ],
    out_shape=[
        jax.ShapeDtypeStruct((N_BLOCKS, 1, BLK_IN), jnp.int32),
        jax.ShapeDtypeStruct((1, 1), jnp.float32),
    ],
)


_SC_WORKERS = 32
_ROWS_PER_W = N_INPUTS // _SC_WORKERS


@functools.lru_cache(maxsize=1)
def _make_sc_gather():
    @functools.partial(
        pl.kernel,
        mesh=plsc.VectorSubcoreMesh(core_axis_name="c", subcore_axis_name="s"),
        out_type=jax.ShapeDtypeStruct((N_INPUTS, DIM), jnp.float32),
        scratch_types=[
            pltpu.VMEM((_ROWS_PER_W,), jnp.int32),
            pltpu.VMEM((_ROWS_PER_W, DIM), jnp.float32),
            pltpu.SemaphoreType.DMA,
        ],
        compiler_params=pltpu.CompilerParams(use_tc_tiling_on_sc=False),
    )
    def _sc_gather(table_hbm, idx_hbm, out_hbm, idx_v, rows_v, sem):
        wid = lax.axis_index("s") * 2 + lax.axis_index("c")
        base = wid * _ROWS_PER_W
        pltpu.sync_copy(idx_hbm.at[pl.ds(base, _ROWS_PER_W)], idx_v)
        pltpu.async_copy(table_hbm.at[idx_v], rows_v, sem).wait()
        pltpu.sync_copy(rows_v, out_hbm.at[pl.ds(base, _ROWS_PER_W)])

    return _sc_gather


def kernel(inputs, embeddings):
    input_shape = inputs.shape
    flat = inputs.reshape(-1, DIM)
    xn = jnp.sum(flat ** 2, axis=1)
    en = jnp.sum(embeddings ** 2, axis=1, keepdims=True)
    idx3, loss = _argmin_call(flat.T.astype(jnp.bfloat16), -2.0 * embeddings,
                              en, xn.reshape(N_BLOCKS, 1, BLK_IN))
    idx = idx3.reshape(N_INPUTS)
    quant = _make_sc_gather()(embeddings, idx).reshape(input_shape)
    return (quant, loss[0, 0], idx)


# speedup vs baseline: 1.5363x; 1.1777x over previous
import functools

import jax
import jax.numpy as jnp
from jax import lax
from jax.experimental import pallas as pl
from jax.experimental.pallas import tpu as pltpu
from jax.experimental.pallas import tpu_sc as plsc

N_CODES = 8192
DIM = 32
N_INPUTS = 8192
BLK_IN = 8192
BLK_CODE = 1024
N_BLOCKS = N_INPUTS // BLK_IN
N_CHUNKS = N_CODES // BLK_CODE


def _argmin_body(xt_ref, emb_ref, en_ref, xn_ref, idx_ref, loss_ref):
    xt = xt_ref[...]
    xn = xn_ref[0]

    def tile_minarg(t):
        e2 = emb_ref[pl.ds(t * BLK_CODE, BLK_CODE), :]
        en = en_ref[pl.ds(t * BLK_CODE, BLK_CODE), :]
        mm = lax.dot_general(e2, xt, (((1,), (0,)), ((), ())),
                             preferred_element_type=jnp.float32)
        d2 = (xn + en) + mm
        tmin2 = jnp.min(d2, axis=0, keepdims=True)
        tidx = jnp.argmin(d2, axis=0)[None, :] + t * BLK_CODE
        tmin = jnp.sqrt(jnp.maximum(tmin2, 0.0))
        return tmin, tidx

    def chunk_step(c, carry):
        bq, bi, fv = carry
        tpc = 2048 // BLK_CODE
        cmin, cidx = tile_minarg(tpc * c)
        for u in range(1, tpc):
            tmin, tidx = tile_minarg(tpc * c + u)
            b = tmin < cmin
            cmin = jnp.where(b, tmin, cmin)
            cidx = jnp.where(b, tidx, cidx)
        upd = cmin < bq
        bq = jnp.where(upd, cmin.astype(jnp.bfloat16).astype(jnp.float32), bq)
        bi = jnp.where(upd, cidx, bi)
        return (bq, bi, jnp.minimum(fv, cmin))

    init = (jnp.full((1, BLK_IN), jnp.inf, jnp.float32),
            jnp.zeros((1, BLK_IN), jnp.int32),
            jnp.full((1, BLK_IN), jnp.inf, jnp.float32))
    _, best_i, best_d = lax.fori_loop(0, N_CHUNKS // 2, chunk_step, init)
    idx_ref[0] = best_i

    pid = pl.program_id(0)

    @pl.when(pid == 0)
    def _init():
        loss_ref[...] = jnp.zeros((1, 1), jnp.float32)

    loss_ref[...] = loss_ref[...] + jnp.sum(best_d * best_d).reshape(1, 1)

    @pl.when(pid == N_BLOCKS - 1)
    def _finish():
        loss_ref[...] = loss_ref[...] * (1.0 / (N_INPUTS * DIM))


_argmin_call = pl.pallas_call(
    _argmin_body,
    grid=(N_BLOCKS,),
    in_specs=[
        pl.BlockSpec((DIM, BLK_IN), lambda i: (0, i)),
        pl.BlockSpec((N_CODES, DIM), lambda i: (0, 0)),
        pl.BlockSpec((N_CODES, 1), lambda i: (0, 0)),
        pl.BlockSpec((1, 1, BLK_IN), lambda i: (i, 0, 0)),
    ],
    out_specs=[
        pl.BlockSpec((1, 1, BLK_IN), lambda i: (i, 0, 0)),
        pl.BlockSpec((1, 1), lambda i: (0, 0)),
    ],
    out_shape=[
        jax.ShapeDtypeStruct((N_BLOCKS, 1, BLK_IN), jnp.int32),
        jax.ShapeDtypeStruct((1, 1), jnp.float32),
    ],
)


_SC_WORKERS = 32
_ROWS_PER_W = N_INPUTS // _SC_WORKERS


@functools.lru_cache(maxsize=1)
def _make_sc_gather():
    @functools.partial(
        pl.kernel,
        mesh=plsc.VectorSubcoreMesh(core_axis_name="c", subcore_axis_name="s"),
        out_type=jax.ShapeDtypeStruct((N_INPUTS, DIM), jnp.float32),
        scratch_types=[
            pltpu.VMEM((_ROWS_PER_W,), jnp.int32),
            pltpu.VMEM((_ROWS_PER_W, DIM), jnp.float32),
            pltpu.SemaphoreType.DMA,
        ],
        compiler_params=pltpu.CompilerParams(use_tc_tiling_on_sc=False),
    )
    def _sc_gather(table_hbm, idx_hbm, out_hbm, idx_v, rows_v, sem):
        wid = lax.axis_index("s") * 2 + lax.axis_index("c")
        base = wid * _ROWS_PER_W
        pltpu.sync_copy(idx_hbm.at[pl.ds(base, _ROWS_PER_W)], idx_v)
        pltpu.async_copy(table_hbm.at[idx_v], rows_v, sem).wait()
        pltpu.sync_copy(rows_v, out_hbm.at[pl.ds(base, _ROWS_PER_W)])

    return _sc_gather


def kernel(inputs, embeddings):
    input_shape = inputs.shape
    flat = inputs.reshape(-1, DIM)
    xn = jnp.sum(flat ** 2, axis=1)
    en = jnp.sum(embeddings ** 2, axis=1, keepdims=True)
    idx3, loss = _argmin_call(flat.T.astype(jnp.bfloat16), -2.0 * embeddings,
                              en, xn.reshape(N_BLOCKS, 1, BLK_IN))
    idx = idx3.reshape(N_INPUTS)
    quant = _make_sc_gather()(embeddings, idx).reshape(input_shape)
    return (quant, loss[0, 0], idx)
